# baseline (speedup 1.0000x reference)
import functools

import jax
import jax.numpy as jnp
from jax import lax
from jax.experimental import pallas as pl
from jax.experimental.pallas import tpu as pltpu
from jax.experimental.pallas import tpu_sc as plsc

N = 10000
D = 128
E = 320000
CNTW = 16
N_CNT = 10048
NC, NS = 2, 16
NW = NC * NS
CH = 128
TILES = E // CH
TPW = TILES // NW
ROWS_PER_TILE = N // NS
CNT_ROWS_PER_TILE = N_CNT // NS


def _sc_gather_scatter(x, et):
    mesh = plsc.VectorSubcoreMesh(core_axis_name="c", subcore_axis_name="s")

    @functools.partial(
        pl.kernel,
        out_type=(jax.ShapeDtypeStruct((NC, N, D), jnp.float32),
                  jax.ShapeDtypeStruct((NC, N_CNT, CNTW), jnp.float32)),
        mesh=mesh,
        scratch_types=[
            pltpu.VMEM((2, CH), jnp.int32),
            pltpu.VMEM((2, CH), jnp.int32),
            pltpu.VMEM((CH, D), jnp.float32),
            pltpu.VMEM((CH, D), jnp.float32),
            pltpu.VMEM((CH, CNTW), jnp.float32),
            pltpu.VMEM_SHARED((N, D), jnp.float32),
            pltpu.VMEM_SHARED((N_CNT, CNTW), jnp.float32),
            pltpu.SemaphoreType.DMA,
            pltpu.SemaphoreType.DMA,
            pltpu.SemaphoreType.DMA,
            pltpu.SemaphoreType.DMA,
            pltpu.SemaphoreType.DMA,
            pltpu.SemaphoreType.DMA,
            pltpu.SemaphoreType.DMA,
            pltpu.SemaphoreType.DMA,
        ],
        compiler_params=pltpu.CompilerParams(use_tc_tiling_on_sc=False,
                                             disable_bounds_checks=True),
    )
    def k(x_hbm, e_hbm, outf_hbm, outc_hbm,
          ib0, ib1, buf0, buf1, ones_v, accf, accc,
          gs0, gs1, fs0, fs1, is0, is1, cs0, cs1):
        core = lax.axis_index("c")
        sub = lax.axis_index("s")
        wid = core * NS + sub
        base = sub * ROWS_PER_TILE
        cbase = sub * CNT_ROWS_PER_TILE
        t0 = wid * TPW

        zeros16 = jnp.zeros((16,), jnp.float32)

        @pl.loop(0, CH)
        def _(r):
            ones_v[r, pl.ds(0, 16)] = zeros16

            @pl.loop(0, D // 16)
            def _(j):
                buf0[r, pl.ds(j * 16, 16)] = zeros16

        @pl.loop(0, ROWS_PER_TILE // CH)
        def _(b):
            pltpu.sync_copy(buf0, accf.at[pl.ds(base + b * CH, CH)])

        rem = ROWS_PER_TILE % CH
        if rem:
            pltpu.sync_copy(buf0.at[pl.ds(0, rem)],
                            accf.at[pl.ds(base + ROWS_PER_TILE - rem, rem)])

        @pl.loop(0, CNT_ROWS_PER_TILE // CH)
        def _(b):
            pltpu.sync_copy(ones_v, accc.at[pl.ds(cbase + b * CH, CH)])

        crem = CNT_ROWS_PER_TILE % CH
        if crem:
            pltpu.sync_copy(
                ones_v.at[pl.ds(0, crem)],
                accc.at[pl.ds(cbase + CNT_ROWS_PER_TILE - crem, crem)])

        ones16 = jnp.ones((16,), jnp.float32)

        @pl.loop(0, CH)
        def _(r):
            ones_v[r, pl.ds(0, 16)] = ones16

        plsc.subcore_barrier()

        pltpu.sync_copy(e_hbm.at[t0], ib0)
        pltpu.async_copy(e_hbm.at[t0 + 1], ib1, is1).wait()
        pltpu.async_copy(x_hbm.at[ib0.at[1]], buf0, gs0)
        pltpu.async_copy(x_hbm.at[ib1.at[1]], buf1, gs1)

        @pl.loop(0, TPW, step=2)
        def _(k):
            pltpu.make_async_copy(x_hbm.at[ib0.at[1]], buf0, gs0).wait()
            f0 = pltpu.async_copy(buf0, accf.at[ib0.at[0]], fs0, add=True)
            c0 = pltpu.async_copy(ones_v, accc.at[ib0.at[0]], cs0, add=True)
            pltpu.make_async_copy(x_hbm.at[ib1.at[1]], buf1, gs1).wait()
            f1 = pltpu.async_copy(buf1, accf.at[ib1.at[0]], fs1, add=True)
            c1 = pltpu.async_copy(ones_v, accc.at[ib1.at[0]], cs1, add=True)
            f0.wait()
            c0.wait()
            k2 = jnp.minimum(t0 + k + 2, t0 + TPW - 1)
            i0 = pltpu.async_copy(e_hbm.at[k2], ib0, is0)
            f1.wait()
            c1.wait()
            k3 = jnp.minimum(t0 + k + 3, t0 + TPW - 1)
            i1 = pltpu.async_copy(e_hbm.at[k3], ib1, is1)
            i0.wait()
            pltpu.async_copy(x_hbm.at[ib0.at[1]], buf0, gs0)
            i1.wait()
            pltpu.async_copy(x_hbm.at[ib1.at[1]], buf1, gs1)

        pltpu.make_async_copy(x_hbm.at[ib0.at[1]], buf0, gs0).wait()
        pltpu.make_async_copy(x_hbm.at[ib1.at[1]], buf1, gs1).wait()

        @pl.when(wid < TILES - NW * TPW)
        def _():
            pltpu.sync_copy(e_hbm.at[NW * TPW + wid], ib0)
            pltpu.async_copy(x_hbm.at[ib0.at[1]], buf0, gs0).wait()
            pltpu.sync_copy(buf0, accf.at[ib0.at[0]], add=True)
            pltpu.sync_copy(ones_v, accc.at[ib0.at[0]], add=True)

        plsc.subcore_barrier()

        pltpu.sync_copy(accf.at[pl.ds(base, ROWS_PER_TILE)],
                        outf_hbm.at[core, pl.ds(base, ROWS_PER_TILE)])
        pltpu.sync_copy(accc.at[pl.ds(cbase, CNT_ROWS_PER_TILE)],
                        outc_hbm.at[core, pl.ds(cbase, CNT_ROWS_PER_TILE)])

    return k(x, et)


def _tc_finish(pf, cnt, x, wn_t, wr_t, bias2):
    blk = 2000

    def body(p_ref, c_ref, x_ref, wn_ref, wr_ref, b_ref, o_ref):
        s = p_ref[0] + p_ref[1]
        aggr = s / jnp.maximum(c_ref[...], 1.0)
        out = jnp.dot(aggr, wn_ref[...], preferred_element_type=jnp.float32)
        out = out + jnp.dot(x_ref[...], wr_ref[...],
                            preferred_element_type=jnp.float32)
        o_ref[...] = out + b_ref[...]

    return pl.pallas_call(
        body,
        grid=(N // blk,),
        in_specs=[
            pl.BlockSpec((NC, blk, D), lambda i: (0, i, 0)),
            pl.BlockSpec((blk, 1), lambda i: (i, 0)),
            pl.BlockSpec((blk, D), lambda i: (i, 0)),
            pl.BlockSpec((D, D), lambda i: (0, 0)),
            pl.BlockSpec((D, D), lambda i: (0, 0)),
            pl.BlockSpec((1, D), lambda i: (0, 0)),
        ],
        out_specs=pl.BlockSpec((blk, D), lambda i: (i, 0)),
        out_shape=jax.ShapeDtypeStruct((N, D), jnp.float32),
    )(pf, cnt, x, wn_t, wr_t, bias2)


def kernel(x, edge_index, W_neigh, W_root, bias):
    et = (edge_index.astype(jnp.int32)
          .reshape(2, TILES, CH).transpose(1, 0, 2))
    pf, pc = _sc_gather_scatter(x, et)
    cnt = (pc[0] + pc[1])[:N, :1]
    return _tc_finish(pf, cnt, x, W_neigh.T, W_root.T, bias.reshape(1, D))


# speedup vs baseline: 1.0908x; 1.0908x over previous
import functools

import jax
import jax.numpy as jnp
from jax import lax
from jax.experimental import pallas as pl
from jax.experimental.pallas import tpu as pltpu
from jax.experimental.pallas import tpu_sc as plsc

N = 10000
D = 128
E = 320000
CNTW = 16
N_CNT = 10048
NC, NS = 2, 16
NW = NC * NS
CH = 40
NCH = 250
ROWS_PER_TILE = N // NS
CNT_ROWS_PER_TILE = N_CNT // NS


def _sc_gather_scatter(x, e_r):
    mesh = plsc.VectorSubcoreMesh(core_axis_name="c", subcore_axis_name="s")

    @functools.partial(
        pl.kernel,
        out_type=(jax.ShapeDtypeStruct((NC, N, D), jnp.float32),
                  jax.ShapeDtypeStruct((NC, N_CNT, CNTW), jnp.float32)),
        mesh=mesh,
        scratch_types=[
            pltpu.VMEM((NCH, 1, CH), jnp.int32),
            pltpu.VMEM((NCH, 1, CH), jnp.int32),
            pltpu.VMEM((CH, D), jnp.float32),
            pltpu.VMEM((CH, D), jnp.float32),
            pltpu.VMEM((CH, D), jnp.float32),
            pltpu.VMEM((CH, CNTW), jnp.float32),
            pltpu.VMEM_SHARED((N, D), jnp.float32),
            pltpu.VMEM_SHARED((N_CNT, CNTW), jnp.float32),
            pltpu.SemaphoreType.DMA,
            pltpu.SemaphoreType.DMA,
            pltpu.SemaphoreType.DMA,
            pltpu.SemaphoreType.DMA,
            pltpu.SemaphoreType.DMA,
            pltpu.SemaphoreType.DMA,
            pltpu.SemaphoreType.DMA,
        ],
        compiler_params=pltpu.CompilerParams(use_tc_tiling_on_sc=False,
                                             disable_bounds_checks=True),
    )
    def k(x_hbm, e_hbm, outf_hbm, outc_hbm,
          row_v, col_v, buf0, buf1, buf2, ones_v, accf, accc,
          gs0, gs1, gs2, fs0, fs1, fs2, cs):
        core = lax.axis_index("c")
        sub = lax.axis_index("s")
        wid = core * NS + sub
        base = sub * ROWS_PER_TILE
        cbase = sub * CNT_ROWS_PER_TILE

        zeros16 = jnp.zeros((16,), jnp.float32)

        @pl.loop(0, CH)
        def _(r):
            ones_v[r, pl.ds(0, 16)] = zeros16

            @pl.loop(0, D // 16)
            def _(j):
                buf0[r, pl.ds(j * 16, 16)] = zeros16

        h0 = pltpu.async_copy(e_hbm.at[0, wid], row_v, cs)
        h1 = pltpu.async_copy(e_hbm.at[1, wid], col_v, cs)

        @pl.loop(0, ROWS_PER_TILE // CH)
        def _(b):
            pltpu.sync_copy(buf0, accf.at[pl.ds(base + b * CH, CH)])

        rem = ROWS_PER_TILE % CH
        if rem:
            pltpu.sync_copy(buf0.at[pl.ds(0, rem)],
                            accf.at[pl.ds(base + ROWS_PER_TILE - rem, rem)])

        @pl.loop(0, CNT_ROWS_PER_TILE // CH)
        def _(b):
            pltpu.sync_copy(ones_v, accc.at[pl.ds(cbase + b * CH, CH)])

        crem = CNT_ROWS_PER_TILE % CH
        if crem:
            pltpu.sync_copy(
                ones_v.at[pl.ds(0, crem)],
                accc.at[pl.ds(cbase + CNT_ROWS_PER_TILE - crem, crem)])

        ones16 = jnp.ones((16,), jnp.float32)

        @pl.loop(0, CH)
        def _(r):
            ones_v[r, pl.ds(0, 16)] = ones16

        h0.wait()
        h1.wait()

        plsc.subcore_barrier()

        pltpu.async_copy(x_hbm.at[col_v.at[0, 0]], buf0, gs0)
        pltpu.async_copy(x_hbm.at[col_v.at[1, 0]], buf1, gs1)
        pltpu.async_copy(x_hbm.at[col_v.at[2, 0]], buf2, gs2)

        @pl.loop(0, NCH - 1, step=3)
        def _(c):
            pltpu.make_async_copy(x_hbm.at[col_v.at[c, 0]], buf0, gs0).wait()
            f0 = pltpu.async_copy(buf0, accf.at[row_v.at[c, 0]], fs0,
                                  add=True)
            pltpu.make_async_copy(x_hbm.at[col_v.at[c + 1, 0]], buf1,
                                  gs1).wait()
            f1 = pltpu.async_copy(buf1, accf.at[row_v.at[c + 1, 0]], fs1,
                                  add=True)
            pltpu.make_async_copy(x_hbm.at[col_v.at[c + 2, 0]], buf2,
                                  gs2).wait()
            f2 = pltpu.async_copy(buf2, accf.at[row_v.at[c + 2, 0]], fs2,
                                  add=True)
            cc0 = pltpu.async_copy(ones_v, accc.at[row_v.at[c, 0]], cs,
                                   add=True)
            cc1 = pltpu.async_copy(ones_v, accc.at[row_v.at[c + 1, 0]], cs,
                                   add=True)
            cc2 = pltpu.async_copy(ones_v, accc.at[row_v.at[c + 2, 0]], cs,
                                   add=True)
            f0.wait()
            pltpu.async_copy(x_hbm.at[col_v.at[c + 3, 0]], buf0, gs0)
            f1.wait()
            c4 = jnp.minimum(c + 4, NCH - 1)
            pltpu.async_copy(x_hbm.at[col_v.at[c4, 0]], buf1, gs1)
            f2.wait()
            c5 = jnp.minimum(c + 5, NCH - 1)
            pltpu.async_copy(x_hbm.at[col_v.at[c5, 0]], buf2, gs2)
            cc0.wait()
            cc1.wait()
            cc2.wait()

        pltpu.make_async_copy(x_hbm.at[col_v.at[NCH - 1, 0]], buf0,
                              gs0).wait()
        pltpu.sync_copy(buf0, accf.at[row_v.at[NCH - 1, 0]], add=True)
        pltpu.sync_copy(ones_v, accc.at[row_v.at[NCH - 1, 0]], add=True)
        pltpu.make_async_copy(x_hbm.at[col_v.at[NCH - 1, 0]], buf1,
                              gs1).wait()
        pltpu.make_async_copy(x_hbm.at[col_v.at[NCH - 1, 0]], buf2,
                              gs2).wait()

        plsc.subcore_barrier()

        pltpu.sync_copy(accf.at[pl.ds(base, ROWS_PER_TILE)],
                        outf_hbm.at[core, pl.ds(base, ROWS_PER_TILE)])
        pltpu.sync_copy(accc.at[pl.ds(cbase, CNT_ROWS_PER_TILE)],
                        outc_hbm.at[core, pl.ds(cbase, CNT_ROWS_PER_TILE)])

    return k(x, e_r)


def _tc_finish(pf, cnt, x, wn_t, wr_t, bias2):
    blk = 2000

    def body(p_ref, c_ref, x_ref, wn_ref, wr_ref, b_ref, o_ref):
        s = p_ref[0] + p_ref[1]
        aggr = s / jnp.maximum(c_ref[...], 1.0)
        out = jnp.dot(aggr, wn_ref[...], preferred_element_type=jnp.float32)
        out = out + jnp.dot(x_ref[...], wr_ref[...],
                            preferred_element_type=jnp.float32)
        o_ref[...] = out + b_ref[...]

    return pl.pallas_call(
        body,
        grid=(N // blk,),
        in_specs=[
            pl.BlockSpec((NC, blk, D), lambda i: (0, i, 0)),
            pl.BlockSpec((blk, 1), lambda i: (i, 0)),
            pl.BlockSpec((blk, D), lambda i: (i, 0)),
            pl.BlockSpec((D, D), lambda i: (0, 0)),
            pl.BlockSpec((D, D), lambda i: (0, 0)),
            pl.BlockSpec((1, D), lambda i: (0, 0)),
        ],
        out_specs=pl.BlockSpec((blk, D), lambda i: (i, 0)),
        out_shape=jax.ShapeDtypeStruct((N, D), jnp.float32),
    )(pf, cnt, x, wn_t, wr_t, bias2)


def kernel(x, edge_index, W_neigh, W_root, bias):
    e_r = edge_index.astype(jnp.int32).reshape(2, NW, NCH, 1, CH)
    pf, pc = _sc_gather_scatter(x, e_r)
    cnt = (pc[0] + pc[1])[:N, :1]
    return _tc_finish(pf, cnt, x, W_neigh.T, W_root.T, bias.reshape(1, D))
